# baseline (speedup 1.0000x reference)
import jax
import jax.numpy as jnp
from jax import lax
from jax.experimental import pallas as pl
from jax.experimental.pallas import tpu as pltpu
from jax.experimental.pallas import tpu_sc as plsc

B = 16384
L = 50
D = 64
M0 = 16
M1 = 32
MTOT = M0 + M1

NW = 32
ROWS_W = B // NW


def _make_sc_hist(nbins, stride, row_chunk):
    cnt_w = row_chunk * stride

    def body(idx_hbm, counts_hbm, idx_v, cnt_v, sem):
        c = lax.axis_index("c")
        s = lax.axis_index("s")
        wid = s * 2 + c
        base = wid * ROWS_W
        cp = pltpu.async_copy(idx_hbm.at[pl.ds(base, ROWS_W), :], idx_v, sem)

        zeros16 = jnp.zeros((16,), jnp.float32)
        lane = lax.iota(jnp.int32, 16)
        ones16 = jnp.ones((16,), jnp.float32)

        cp.wait()

        def chunk_body(ch, carry):
            @plsc.parallel_loop(0, row_chunk, unroll=8)
            def zero_body(i):
                for j in range(nbins // 16):
                    cnt_v[pl.ds(i * stride + j * 16, 16)] = zeros16

            @plsc.parallel_loop(0, row_chunk // 16)
            def g_body(g):
                rows = ch * row_chunk + g * 16 + lane
                trow = (g * 16 + lane) * stride

                @plsc.parallel_loop(0, L, unroll=10)
                def l_body(l):
                    cols = jnp.full((16,), l, jnp.int32)
                    v = plsc.load_gather(idx_v, [rows, cols])
                    b = lax.bitwise_and(v, nbins - 1)
                    plsc.addupdate_scatter(cnt_v, [trow + b], ones16)

            pltpu.sync_copy(
                cnt_v,
                counts_hbm.at[pl.ds((base + ch * row_chunk) * stride, cnt_w)],
            )
            return carry

        lax.fori_loop(0, ROWS_W // row_chunk, chunk_body, 0)

    def call(idx):
        return pl.kernel(
            body,
            out_type=jax.ShapeDtypeStruct((B * stride,), jnp.float32),
            mesh=plsc.VectorSubcoreMesh(
                core_axis_name="c", subcore_axis_name="s"),
            compiler_params=pltpu.CompilerParams(
                needs_layout_passes=False, use_tc_tiling_on_sc=True),
            scratch_types=[
                pltpu.VMEM((ROWS_W, L), jnp.int32),
                pltpu.VMEM((cnt_w,), jnp.float32),
                pltpu.SemaphoreType.DMA,
            ],
        )(idx)

    return call


C1_STRIDE = 128
_sc_hist0 = _make_sc_hist(M0, M0, ROWS_W)
_sc_hist1 = _make_sc_hist(M1, C1_STRIDE, ROWS_W // 2)


TC_ROWS = 2048
NBLK = B // TC_ROWS


def _tc_matmul_body(c0_ref, c1_ref, w_ref, pred_ref, loss_ref):
    i = pl.program_id(0)
    c = jnp.concatenate([c0_ref[...], c1_ref[:, :M1]], axis=1)
    p = jnp.dot(
        c,
        w_ref[...],
        preferred_element_type=jnp.float32,
        precision=lax.Precision.HIGHEST,
    )
    pred_ref[...] = p

    @pl.when(i == 0)
    def _():
        loss_ref[...] = jnp.zeros((1, 1), jnp.float32)

    loss_ref[...] += jnp.sum(p).reshape(1, 1)

    @pl.when(i == NBLK - 1)
    def _():
        loss_ref[...] = loss_ref[...] / (B * 2 * D)


def _tc_matmul(c0, c1, w):
    return pl.pallas_call(
        _tc_matmul_body,
        grid=(NBLK,),
        in_specs=[
            pl.BlockSpec((TC_ROWS, M0), lambda i: (i, 0)),
            pl.BlockSpec((TC_ROWS, C1_STRIDE), lambda i: (i, 0)),
            pl.BlockSpec((MTOT, 2 * D), lambda i: (0, 0)),
        ],
        out_specs=[
            pl.BlockSpec((TC_ROWS, 2 * D), lambda i: (i, 0)),
            pl.BlockSpec((1, 1), lambda i: (0, 0)),
        ],
        out_shape=[
            jax.ShapeDtypeStruct((B, 2 * D), jnp.float32),
            jax.ShapeDtypeStruct((1, 1), jnp.float32),
        ],
    )(c0, c1, w)


def kernel(indices_0, indices_1, table_0, table_1):
    counts0 = _sc_hist0(indices_0).reshape(B, M0)
    counts1 = _sc_hist1(indices_1).reshape(B, C1_STRIDE)
    w = (
        jnp.zeros((MTOT, 2 * D), table_0.dtype)
        .at[:M0, :D].set(table_0)
        .at[M0:, D:].set(table_1)
    )
    pred, loss = _tc_matmul(counts0, counts1, w)
    return loss[0, 0], pred


# speedup vs baseline: 1.1699x; 1.1699x over previous
import jax
import jax.numpy as jnp
from jax import lax
from jax.experimental import pallas as pl
from jax.experimental.pallas import tpu as pltpu
from jax.experimental.pallas import tpu_sc as plsc

B = 16384
L = 50
D = 64
M0 = 16
M1 = 32
MTOT = M0 + M1

NW = 32
ROWS_W = B // NW
GROUPS = ROWS_W // 16
IDX_W = ROWS_W * L


def _make_sc_hist_tiled(nbins, stride):
    cnt_w = ROWS_W * stride

    def body(idx_hbm, counts_hbm, idx_v, cnt_v, sem):
        c = lax.axis_index("c")
        s = lax.axis_index("s")
        wid = s * 2 + c
        base = wid * ROWS_W
        cp = pltpu.async_copy(idx_hbm.at[pl.ds(base, ROWS_W), :], idx_v, sem)

        zeros16 = jnp.zeros((16,), jnp.float32)
        lane = lax.iota(jnp.int32, 16)
        ones16 = jnp.ones((16,), jnp.float32)

        @plsc.parallel_loop(0, ROWS_W, unroll=8)
        def zero_body(i):
            for j in range(nbins // 16):
                cnt_v[pl.ds(i * stride + j * 16, 16)] = zeros16

        cp.wait()

        @plsc.parallel_loop(0, GROUPS)
        def g_body(g):
            rows = g * 16 + lane
            trow = rows * stride

            @plsc.parallel_loop(0, L, unroll=10)
            def l_body(l):
                cols = jnp.full((16,), l, jnp.int32)
                v = plsc.load_gather(idx_v, [rows, cols])
                b = lax.bitwise_and(v, nbins - 1)
                plsc.addupdate_scatter(cnt_v, [trow + b], ones16)

        pltpu.sync_copy(cnt_v, counts_hbm.at[pl.ds(base * stride, cnt_w)])

    def call(idx):
        return pl.kernel(
            body,
            out_type=jax.ShapeDtypeStruct((B * stride,), jnp.float32),
            mesh=plsc.VectorSubcoreMesh(
                core_axis_name="c", subcore_axis_name="s"),
            compiler_params=pltpu.CompilerParams(
                needs_layout_passes=False, use_tc_tiling_on_sc=True),
            scratch_types=[
                pltpu.VMEM((ROWS_W, L), jnp.int32),
                pltpu.VMEM((cnt_w,), jnp.float32),
                pltpu.SemaphoreType.DMA,
            ],
        )(idx)

    return call


def _make_sc_hist_flat(nbins, stride):
    cnt_w = ROWS_W * stride

    def body(idx_hbm, counts_hbm, idx_v, cnt_v, sem):
        c = lax.axis_index("c")
        s = lax.axis_index("s")
        wid = s * 2 + c
        cp = pltpu.async_copy(idx_hbm.at[pl.ds(wid * IDX_W, IDX_W)], idx_v, sem)

        zeros16 = jnp.zeros((16,), jnp.float32)
        lane = lax.iota(jnp.int32, 16)
        ones16 = jnp.ones((16,), jnp.float32)

        @plsc.parallel_loop(0, ROWS_W, unroll=8)
        def zero_body(i):
            for j in range(nbins // 16):
                cnt_v[pl.ds(i * stride + j * 16, 16)] = zeros16

        cp.wait()

        @plsc.parallel_loop(0, GROUPS)
        def g_body(g):
            rows = g * 16 + lane
            addr_base = rows * L
            trow = rows * stride

            @plsc.parallel_loop(0, L, unroll=10)
            def l_body(l):
                v = plsc.load_gather(idx_v, [addr_base + l])
                b = lax.bitwise_and(v, nbins - 1)
                plsc.addupdate_scatter(cnt_v, [trow + b], ones16)

        pltpu.sync_copy(
            cnt_v, counts_hbm.at[pl.ds(wid * cnt_w, cnt_w)])

    def call(idx_flat):
        return pl.kernel(
            body,
            out_type=jax.ShapeDtypeStruct((B * stride,), jnp.float32),
            mesh=plsc.VectorSubcoreMesh(
                core_axis_name="c", subcore_axis_name="s"),
            compiler_params=pltpu.CompilerParams(needs_layout_passes=False),
            scratch_types=[
                pltpu.VMEM((IDX_W,), jnp.int32),
                pltpu.VMEM((cnt_w,), jnp.float32),
                pltpu.SemaphoreType.DMA,
            ],
        )(idx_flat)

    return call


C1_STRIDE = 128
_sc_hist0 = _make_sc_hist_tiled(M0, M0)
_sc_hist1 = _make_sc_hist_flat(M1, C1_STRIDE)


TC_ROWS = 2048
NBLK = B // TC_ROWS


def _tc_matmul_body(c0_ref, c1_ref, w_ref, pred_ref, loss_ref):
    i = pl.program_id(0)
    c = jnp.concatenate([c0_ref[...], c1_ref[:, :M1]], axis=1)
    p = jnp.dot(
        c,
        w_ref[...],
        preferred_element_type=jnp.float32,
        precision=lax.Precision.HIGHEST,
    )
    pred_ref[...] = p

    @pl.when(i == 0)
    def _():
        loss_ref[...] = jnp.zeros((1, 1), jnp.float32)

    loss_ref[...] += jnp.sum(p).reshape(1, 1)

    @pl.when(i == NBLK - 1)
    def _():
        loss_ref[...] = loss_ref[...] / (B * 2 * D)


def _tc_matmul(c0, c1, w):
    return pl.pallas_call(
        _tc_matmul_body,
        grid=(NBLK,),
        in_specs=[
            pl.BlockSpec((TC_ROWS, M0), lambda i: (i, 0)),
            pl.BlockSpec((TC_ROWS, C1_STRIDE), lambda i: (i, 0)),
            pl.BlockSpec((MTOT, 2 * D), lambda i: (0, 0)),
        ],
        out_specs=[
            pl.BlockSpec((TC_ROWS, 2 * D), lambda i: (i, 0)),
            pl.BlockSpec((1, 1), lambda i: (0, 0)),
        ],
        out_shape=[
            jax.ShapeDtypeStruct((B, 2 * D), jnp.float32),
            jax.ShapeDtypeStruct((1, 1), jnp.float32),
        ],
    )(c0, c1, w)


def kernel(indices_0, indices_1, table_0, table_1):
    counts0 = _sc_hist0(indices_0).reshape(B, M0)
    counts1 = _sc_hist1(indices_1.reshape(-1)).reshape(B, C1_STRIDE)
    w = (
        jnp.zeros((MTOT, 2 * D), table_0.dtype)
        .at[:M0, :D].set(table_0)
        .at[M0:, D:].set(table_1)
    )
    pred, loss = _tc_matmul(counts0, counts1, w)
    return loss[0, 0], pred
